# baseline (speedup 1.0000x reference)
import functools

import jax
import jax.numpy as jnp
from jax import lax
from jax.experimental import pallas as pl
from jax.experimental.pallas import tpu as pltpu
from jax.experimental.pallas import tpu_sc as plsc

B = 1024
MAXLEN = 1024
EMB = 128
VOCAB = 21

NC = 2
NS = 16
NW = NC * NS
NTOK = B * MAXLEN
CH = 128
NBUF = 4

TR = 32
SC_TOK = 524288
assert SC_TOK % (NW * CH * NBUF) == 0 and (NTOK - SC_TOK) % (TR * CH) == 0


@functools.cache
def _make_emb_sc(sc_tok):
    nch = sc_tok // (NW * CH)
    mesh = plsc.VectorSubcoreMesh(
        core_axis_name="c", subcore_axis_name="s", num_cores=NC, num_subcores=NS
    )

    def body(tok_hbm, table_hbm, out_hbm, tok_v, rows_v, table_v, gs, os):
        wid = lax.axis_index("s") * NC + lax.axis_index("c")
        chunk0 = wid * nch

        @pl.when(lax.axis_index("s") == 0)
        def _():
            pltpu.sync_copy(table_hbm, table_v)

        pltpu.sync_copy(tok_hbm.at[pl.ds(chunk0, nch)], tok_v)
        plsc.subcore_barrier()

        def start_gather(c, b):
            pltpu.async_copy(table_v.at[tok_v.at[c]], rows_v.at[b], gs[b])

        def wait_gather(c, b):
            pltpu.make_async_copy(table_v.at[tok_v.at[c]], rows_v.at[b], gs[b]).wait()

        def start_store(c, b):
            pltpu.async_copy(
                rows_v.at[b], out_hbm.at[pl.ds((chunk0 + c) * CH, CH)], os[b]
            )

        def wait_store(b):
            pltpu.make_async_copy(
                rows_v.at[b], out_hbm.at[pl.ds(chunk0 * CH, CH)], os[b]
            ).wait()

        for b in range(NBUF):
            start_gather(b, b)

        @pl.loop(0, nch // NBUF - 1)
        def _(i):
            c0 = i * NBUF
            for b in range(NBUF):
                wait_gather(c0 + b, b)
                start_store(c0 + b, b)
            for b in range(NBUF):
                wait_store(b)
                start_gather(c0 + NBUF + b, b)

        c0 = nch - NBUF
        for b in range(NBUF):
            wait_gather(c0 + b, b)
            start_store(c0 + b, b)
        for b in range(NBUF):
            wait_store(b)

    return pl.kernel(
        body,
        out_type=jax.ShapeDtypeStruct((sc_tok, EMB), jnp.float32),
        mesh=mesh,
        scratch_types=[
            pltpu.VMEM((nch, CH), jnp.int32),
            pltpu.VMEM((NBUF, CH, EMB), jnp.float32),
            pltpu.VMEM_SHARED((VOCAB, EMB), jnp.float32),
            [pltpu.SemaphoreType.DMA] * NBUF,
            [pltpu.SemaphoreType.DMA] * NBUF,
        ],
    )


def _onehot_body(tok_ref, table_ref, out_ref):
    tok = tok_ref[...]
    oh = tok[:, :, None] == lax.broadcasted_iota(jnp.int32, (1, 1, 128), 2)
    out_ref[...] = lax.dot_general(
        oh.astype(jnp.float32),
        table_ref[...],
        (((2,), (0,)), ((), ())),
        preferred_element_type=jnp.float32,
    )


def _make_emb_tc(tc_tok, row0):
    return pl.pallas_call(
        _onehot_body,
        grid=(tc_tok // (TR * CH),),
        in_specs=[
            pl.BlockSpec((TR, CH), lambda i: (row0 // TR + i, 0)),
            pl.BlockSpec((CH, EMB), lambda i: (0, 0)),
        ],
        out_specs=pl.BlockSpec((TR, CH, EMB), lambda i: (i, 0, 0)),
        out_shape=jax.ShapeDtypeStruct((tc_tok // CH, CH, EMB), jnp.float32),
    )


def _count_body(tok_ref, out_ref):
    t = tok_ref[...].reshape(8, 128, MAXLEN)
    out_ref[...] = jnp.sum((t != 0).astype(jnp.int32), axis=2)


_count_tc = pl.pallas_call(
    _count_body,
    out_shape=jax.ShapeDtypeStruct((8, 128), jnp.int32),
)


def kernel(tokens, table):
    tok2d = tokens.reshape(NTOK // CH, CH)
    table_pad = jnp.pad(table, ((0, CH - VOCAB), (0, 0)))
    sc_part = _make_emb_sc(SC_TOK)(tok2d, table)
    tc_tok = NTOK - SC_TOK
    tc_part = _make_emb_tc(tc_tok, SC_TOK // CH)(tok2d, table_pad)
    emb = jnp.concatenate(
        [sc_part, tc_part.reshape(tc_tok, EMB)], axis=0
    ).reshape(B, MAXLEN, EMB)
    pro_lens = _count_tc(tokens).reshape(B)
    return emb, pro_lens


# speedup vs baseline: 1.6027x; 1.6027x over previous
import functools

import jax
import jax.numpy as jnp
from jax import lax
from jax.experimental import pallas as pl
from jax.experimental.pallas import tpu as pltpu
from jax.experimental.pallas import tpu_sc as plsc

B = 1024
MAXLEN = 1024
EMB = 128
VOCAB = 21

NC = 2
NS = 16
NW = NC * NS
NTOK = B * MAXLEN
TOK_PER_W = NTOK // NW
CH = 128
NCH = TOK_PER_W // CH
NBUF = 2
G = 2
NST = NCH // G

@functools.cache
def _make_emb_sc():
    mesh = plsc.VectorSubcoreMesh(
        core_axis_name="c", subcore_axis_name="s", num_cores=NC, num_subcores=NS
    )
    return functools.partial(
        pl.kernel,
        out_type=jax.ShapeDtypeStruct((NTOK, EMB), jnp.float32),
        mesh=mesh,
        scratch_types=[
            pltpu.VMEM((NCH, CH), jnp.int32),
            pltpu.VMEM((NBUF, G * CH, EMB), jnp.float32),
            pltpu.VMEM_SHARED((VOCAB, EMB), jnp.float32),
            [pltpu.SemaphoreType.DMA] * NBUF,
            [pltpu.SemaphoreType.DMA] * NBUF,
        ],
    )(_emb_sc_body)


def _emb_sc_body(tok_hbm, table_hbm, out_hbm, tok_v, rows_v, table_v, gs, os):
    wid = lax.axis_index("s") * NC + lax.axis_index("c")
    chunk0 = wid * NCH

    @pl.when(lax.axis_index("s") == 0)
    def _():
        pltpu.sync_copy(table_hbm, table_v)

    pltpu.sync_copy(tok_hbm.at[pl.ds(chunk0, NCH)], tok_v)
    plsc.subcore_barrier()

    def start_gathers(s, b):
        for g in range(G):
            pltpu.async_copy(
                table_v.at[tok_v.at[s * G + g]],
                rows_v.at[b].at[pl.ds(g * CH, CH)],
                gs[b],
            )

    def wait_gathers(s, b):
        for g in range(G):
            pltpu.make_async_copy(
                table_v.at[tok_v.at[s * G + g]],
                rows_v.at[b].at[pl.ds(g * CH, CH)],
                gs[b],
            ).wait()

    def start_store(s, b):
        pltpu.async_copy(
            rows_v.at[b], out_hbm.at[pl.ds((chunk0 + s * G) * CH, G * CH)], os[b]
        )

    def wait_store(b):
        pltpu.make_async_copy(
            rows_v.at[b], out_hbm.at[pl.ds(chunk0 * CH, G * CH)], os[b]
        ).wait()

    for b in range(NBUF):
        start_gathers(b, b)

    @pl.loop(0, NST // NBUF - 1)
    def _(i):
        s0 = i * NBUF
        for b in range(NBUF):
            wait_gathers(s0 + b, b)
            start_store(s0 + b, b)
        for b in range(NBUF):
            wait_store(b)
            start_gathers(s0 + NBUF + b, b)

    s0 = NST - NBUF
    for b in range(NBUF):
        wait_gathers(s0 + b, b)
        start_store(s0 + b, b)
    for b in range(NBUF):
        wait_store(b)


def _count_body(tok_ref, out_ref):
    t = tok_ref[...].reshape(8, 128, MAXLEN)
    out_ref[...] = jnp.sum((t != 0).astype(jnp.int32), axis=2)


_count_tc = pl.pallas_call(
    _count_body,
    out_shape=jax.ShapeDtypeStruct((8, 128), jnp.int32),
)


def kernel(tokens, table):
    tok2d = tokens.reshape(NTOK // CH, CH)
    emb_flat = _make_emb_sc()(tok2d, table)
    emb = emb_flat.reshape(B, MAXLEN, EMB)
    pro_lens = _count_tc(tokens).reshape(B)
    return emb, pro_lens


# speedup vs baseline: 2.3525x; 1.4679x over previous
import functools

import jax
import jax.numpy as jnp
from jax import lax
from jax.experimental import pallas as pl
from jax.experimental.pallas import tpu as pltpu
from jax.experimental.pallas import tpu_sc as plsc

B = 1024
MAXLEN = 1024
EMB = 128
VOCAB = 21

NC = 2
NS = 16
NW = NC * NS
NTOK = B * MAXLEN
TOK_PER_W = NTOK // NW
CH = 128
NCH = TOK_PER_W // CH

@functools.cache
def _make_emb_sc():
    mesh = plsc.VectorSubcoreMesh(
        core_axis_name="c", subcore_axis_name="s", num_cores=NC, num_subcores=NS
    )
    return functools.partial(
        pl.kernel,
        out_type=jax.ShapeDtypeStruct((NTOK, EMB), jnp.float32),
        mesh=mesh,
        scratch_types=[
            pltpu.VMEM((NCH, CH), jnp.int32),
            pltpu.VMEM((2, CH, EMB), jnp.float32),
            pltpu.VMEM_SHARED((VOCAB, EMB), jnp.float32),
            pltpu.SemaphoreType.DMA,
            pltpu.SemaphoreType.DMA,
        ],
    )(_emb_sc_body)


def _emb_sc_body(tok_hbm, table_hbm, out_hbm, tok_v, rows_v, table_v, gs0, gs1):
    wid = lax.axis_index("s") * NC + lax.axis_index("c")
    chunk0 = wid * NCH

    @pl.when(lax.axis_index("s") == 0)
    def _():
        pltpu.sync_copy(table_hbm, table_v)

    pltpu.sync_copy(tok_hbm.at[pl.ds(chunk0, NCH)], tok_v)
    plsc.subcore_barrier()

    def start_gather(c, buf, sem):
        pltpu.async_copy(table_v.at[tok_v.at[c]], rows_v.at[buf], sem)

    def wait_gather(c, buf, sem):
        pltpu.make_async_copy(table_v.at[tok_v.at[c]], rows_v.at[buf], sem).wait()

    def store(c, buf):
        pltpu.sync_copy(rows_v.at[buf], out_hbm.at[pl.ds((chunk0 + c) * CH, CH)])

    start_gather(0, 0, gs0)
    start_gather(1, 1, gs1)

    @pl.loop(0, NCH // 2 - 1)
    def _(i):
        c = 2 * i
        wait_gather(c, 0, gs0)
        store(c, 0)
        start_gather(c + 2, 0, gs0)
        wait_gather(c + 1, 1, gs1)
        store(c + 1, 1)
        start_gather(c + 3, 1, gs1)

    wait_gather(NCH - 2, 0, gs0)
    store(NCH - 2, 0)
    wait_gather(NCH - 1, 1, gs1)
    store(NCH - 1, 1)


def _count_body(tok_ref, out_ref):
    t = tok_ref[...].reshape(8, 128, MAXLEN)
    out_ref[...] = jnp.sum((t != 0).astype(jnp.int32), axis=2)


_count_tc = pl.pallas_call(
    _count_body,
    out_shape=jax.ShapeDtypeStruct((8, 128), jnp.int32),
)


def kernel(tokens, table):
    tok2d = tokens.reshape(NTOK // CH, CH)
    emb_flat = _make_emb_sc()(tok2d, table)
    emb = emb_flat.reshape(B, MAXLEN, EMB)
    pro_lens = _count_tc(tokens).reshape(B)
    return emb, pro_lens
